# Initial kernel scaffold; baseline (speedup 1.0000x reference)
import jax
import jax.numpy as jnp
from jax.experimental import pallas as pl


def kernel(x, edge_index, fc_in_W, fc_in_b, conv1_W, conv1_b, conv2_W, conv2_b, mlp1_W, mlp1_b, mlp2_W, mlp2_b):
    raise NotImplementedError("write your pallas kernel here")


# speedup vs baseline: 5.4018x; 5.4018x over previous
import functools

import jax
import jax.numpy as jnp
from jax import lax
from jax.experimental import pallas as pl
from jax.experimental.pallas import tpu as pltpu
from jax.experimental.pallas import tpu_sc as plsc

N = 10000
NPAD = 10240
E = 320000
D = 128
H = 128

NC = 2
NS = 16
HN = NPAD // NC
HNP = HN + 8
EPS = E // NS
CH = 80
NCHUNK = EPS // CH
RPT = HN // NS

_MESH = dict(core_axis_name="c", subcore_axis_name="s", num_cores=NC,
             num_subcores=NS)


def _fill(ref, nrows, ncols, val):
    v = jnp.full((16,), val, jnp.float32)

    @pl.loop(0, nrows)
    def _(i):
        for j in range(ncols // 16):
            ref[i, pl.ds(j * 16, 16)] = v


def _remap(idst_v, iloc_v, c):
    base = c * HN

    @pl.loop(0, CH // 16)
    def _(v):
        iv = idst_v[pl.ds(v * 16, 16)]
        rel = iv - base
        ok = (rel >= 0) & (rel < HN)
        iloc_v[pl.ds(v * 16, 16)] = jnp.where(ok, rel, HN)


def _sc_degree(dst):
    @functools.partial(
        pl.kernel,
        out_type=jax.ShapeDtypeStruct((NPAD, 16), jnp.float32),
        mesh=plsc.VectorSubcoreMesh(**_MESH),
        scratch_types=[
            pltpu.VMEM_SHARED((HNP, 16), jnp.float32),
            pltpu.VMEM((CH,), jnp.int32),
            pltpu.VMEM((CH,), jnp.int32),
            pltpu.VMEM((CH, 16), jnp.float32),
            pltpu.VMEM((RPT, 16), jnp.float32),
        ],
    )
    def k(dst_hbm, out_hbm, deg_sh, idst_v, iloc_v, ones_v, zed_v):
        c = lax.axis_index("c")
        s = lax.axis_index("s")
        _fill(ones_v, CH, 16, 1.0)
        _fill(zed_v, RPT, 16, 0.0)
        pltpu.sync_copy(zed_v, deg_sh.at[pl.ds(s * RPT, RPT)])
        plsc.subcore_barrier()

        @pl.loop(0, NCHUNK)
        def _(g):
            base = s * EPS + g * CH
            pltpu.sync_copy(dst_hbm.at[pl.ds(base, CH)], idst_v)
            _remap(idst_v, iloc_v, c)
            pltpu.sync_copy(ones_v, deg_sh.at[iloc_v], add=True)

        plsc.subcore_barrier()
        pltpu.sync_copy(deg_sh.at[pl.ds(s * RPT, RPT)],
                        out_hbm.at[pl.ds(c * HN + s * RPT, RPT)])

    return k(dst)


@functools.cache
def _sc_conv_kernel():
    @functools.partial(
        pl.kernel,
        out_type=jax.ShapeDtypeStruct((NPAD, H), jnp.float32),
        mesh=plsc.VectorSubcoreMesh(**_MESH),
        scratch_types=[
            pltpu.VMEM_SHARED((HNP, H), jnp.float32),
            pltpu.VMEM((CH,), jnp.int32),
            pltpu.VMEM((CH,), jnp.int32),
            pltpu.VMEM((CH,), jnp.int32),
            pltpu.VMEM((CH, H), jnp.float32),
            pltpu.VMEM((RPT, H), jnp.float32),
            pltpu.SemaphoreType.DMA,
        ],
    )
    def k(hs_hbm, src_hbm, dst_hbm, out_hbm, acc_sh, isrc_v, idst_v, iloc_v,
          rows_v, zed_v, sem):
        c = lax.axis_index("c")
        s = lax.axis_index("s")
        _fill(zed_v, RPT, H, 0.0)
        pltpu.sync_copy(zed_v, acc_sh.at[pl.ds(s * RPT, RPT)])
        plsc.subcore_barrier()

        @pl.loop(0, NCHUNK)
        def _(g):
            base = s * EPS + g * CH
            pltpu.sync_copy(src_hbm.at[pl.ds(base, CH)], isrc_v)
            pltpu.sync_copy(dst_hbm.at[pl.ds(base, CH)], idst_v)
            _remap(idst_v, iloc_v, c)
            pltpu.async_copy(hs_hbm.at[isrc_v], rows_v, sem).wait()
            pltpu.sync_copy(rows_v, acc_sh.at[iloc_v], add=True)

        plsc.subcore_barrier()
        pltpu.sync_copy(acc_sh.at[pl.ds(s * RPT, RPT)],
                        out_hbm.at[pl.ds(c * HN + s * RPT, RPT)])

    return k


def _sc_conv(hs, src, dst):
    return _sc_conv_kernel()(hs, src, dst)


NW = NC * NS
EPW = E // NW
NCHUNK_E = EPW // CH


def _sc_edge(a, b, src, dst, w2):
    @functools.partial(
        pl.kernel,
        out_type=jax.ShapeDtypeStruct((E, 16), jnp.float32),
        mesh=plsc.VectorSubcoreMesh(**_MESH),
        scratch_types=[
            pltpu.VMEM((CH,), jnp.int32),
            pltpu.VMEM((CH,), jnp.int32),
            pltpu.VMEM((CH, H), jnp.float32),
            pltpu.VMEM((CH, H), jnp.float32),
            pltpu.VMEM((CH, 16), jnp.float32),
            pltpu.VMEM((H,), jnp.float32),
            pltpu.SemaphoreType.DMA,
            pltpu.SemaphoreType.DMA,
        ],
    )
    def k(a_hbm, b_hbm, src_hbm, dst_hbm, w_hbm, out_hbm, isrc_v, idst_v,
          arow_v, brow_v, part_v, w_v, sema, semb):
        c = lax.axis_index("c")
        s = lax.axis_index("s")
        wid = s * NC + c
        pltpu.sync_copy(w_hbm, w_v)
        wvecs = [w_v[pl.ds(j * 16, 16)] for j in range(H // 16)]

        @pl.loop(0, NCHUNK_E)
        def _(g):
            base = wid * EPW + g * CH
            pltpu.sync_copy(src_hbm.at[pl.ds(base, CH)], isrc_v)
            pltpu.sync_copy(dst_hbm.at[pl.ds(base, CH)], idst_v)
            da = pltpu.async_copy(a_hbm.at[isrc_v], arow_v, sema)
            db = pltpu.async_copy(b_hbm.at[idst_v], brow_v, semb)
            da.wait()
            db.wait()

            @pl.loop(0, CH)
            def _(e):
                acc = jnp.zeros((16,), jnp.float32)
                for j in range(H // 16):
                    av = arow_v[e, pl.ds(j * 16, 16)]
                    bv = brow_v[e, pl.ds(j * 16, 16)]
                    acc = acc + jnp.maximum(av + bv, 0.0) * wvecs[j]
                part_v[e, :] = acc

            pltpu.sync_copy(part_v, out_hbm.at[pl.ds(base, CH)])

    return k(a, b, src, dst, w2)


_BN = 1024


def _tc_linear(x, w, bias=None, scale=None, relu=False):
    in_specs = [
        pl.BlockSpec((_BN, x.shape[1]), lambda i: (i, 0)),
        pl.BlockSpec(w.shape, lambda i: (0, 0)),
    ]
    args = [x, w]
    if bias is not None:
        in_specs.append(pl.BlockSpec((1, H), lambda i: (0, 0)))
        args.append(bias)
    if scale is not None:
        in_specs.append(pl.BlockSpec((_BN, 1), lambda i: (i, 0)))
        args.append(scale)

    def body(*refs):
        x_ref, w_ref, rest = refs[0], refs[1], list(refs[2:-1])
        o_ref = refs[-1]
        y = jnp.dot(x_ref[...], w_ref[...], preferred_element_type=jnp.float32)
        if bias is not None:
            y = y + rest.pop(0)[...]
        if scale is not None:
            y = y * rest.pop(0)[...]
        if relu:
            y = jnp.maximum(y, 0.0)
        o_ref[...] = y

    return pl.pallas_call(
        body,
        grid=(NPAD // _BN,),
        in_specs=in_specs,
        out_specs=pl.BlockSpec((_BN, H), lambda i: (i, 0)),
        out_shape=jax.ShapeDtypeStruct((NPAD, H), jnp.float32),
    )(*args)


def _tc_dinv(deg):
    def body(d_ref, o_ref):
        o_ref[...] = lax.rsqrt(1.0 + d_ref[:, :1])

    return pl.pallas_call(
        body,
        grid=(NPAD // _BN,),
        in_specs=[pl.BlockSpec((_BN, 16), lambda i: (i, 0))],
        out_specs=pl.BlockSpec((_BN, 1), lambda i: (i, 0)),
        out_shape=jax.ShapeDtypeStruct((NPAD, 1), jnp.float32),
    )(deg)


def _tc_merge(acc, hs, dinv, bias):
    def body(a_ref, hs_ref, s_ref, b_ref, o_ref):
        y = s_ref[...] * (a_ref[...] + hs_ref[...]) + b_ref[...]
        o_ref[...] = jnp.maximum(y, 0.0)

    blk = lambda i: (i, 0)
    return pl.pallas_call(
        body,
        grid=(NPAD // _BN,),
        in_specs=[pl.BlockSpec((_BN, H), blk), pl.BlockSpec((_BN, H), blk),
                  pl.BlockSpec((_BN, 1), blk),
                  pl.BlockSpec((1, H), lambda i: (0, 0))],
        out_specs=pl.BlockSpec((_BN, H), blk),
        out_shape=jax.ShapeDtypeStruct((NPAD, H), jnp.float32),
    )(acc, hs, dinv, bias)


_BE = 4000


def _tc_finish(part, b2):
    def body(p_ref, b_ref, o_ref):
        o_ref[...] = jnp.sum(p_ref[...], axis=-1, keepdims=True) + b_ref[...]

    return pl.pallas_call(
        body,
        grid=(E // _BE,),
        in_specs=[pl.BlockSpec((_BE, 16), lambda i: (i, 0)),
                  pl.BlockSpec((1, 1), lambda i: (0, 0))],
        out_specs=pl.BlockSpec((_BE, 1), lambda i: (i, 0)),
        out_shape=jax.ShapeDtypeStruct((E, 1), jnp.float32),
    )(part, b2)


def kernel(x, edge_index, fc_in_W, fc_in_b, conv1_W, conv1_b, conv2_W,
           conv2_b, mlp1_W, mlp1_b, mlp2_W, mlp2_b):
    src = edge_index[0]
    dst = edge_index[1]
    xp = jnp.pad(x, ((0, NPAD - N), (0, 0)))

    h0 = _tc_linear(xp, fc_in_W, bias=fc_in_b.reshape(1, H), relu=True)

    deg = _sc_degree(dst)
    dinv = _tc_dinv(deg)

    hs1 = _tc_linear(h0, conv1_W, scale=dinv)
    acc1 = _sc_conv(hs1, src, dst)
    h1 = _tc_merge(acc1, hs1, dinv, conv1_b.reshape(1, H))

    hs2 = _tc_linear(h1, conv2_W, scale=dinv)
    acc2 = _sc_conv(hs2, src, dst)
    h2 = _tc_merge(acc2, hs2, dinv, conv2_b.reshape(1, H))

    a = _tc_linear(h2, mlp1_W[:H], bias=mlp1_b.reshape(1, H))
    b = _tc_linear(h2, mlp1_W[H:])

    part = _sc_edge(a, b, src, dst, mlp2_W[:, 0])
    logits = _tc_finish(part, mlp2_b.reshape(1, 1))
    return logits[:, 0]


# speedup vs baseline: 9.3397x; 1.7290x over previous
import functools

import jax
import jax.numpy as jnp
from jax import lax
from jax.experimental import pallas as pl
from jax.experimental.pallas import tpu as pltpu
from jax.experimental.pallas import tpu_sc as plsc

N = 10000
NPAD = 10240
E = 320000
D = 128
H = 128
NC = 2
NS = 16
HN = NPAD // NC
HNP = HN + 8
EPS = E // NS
RPT = HN // NS
RPTN = NPAD // NS

CHC = 80
NCH_C = EPS // CHC

_MESH = dict(core_axis_name="c", subcore_axis_name="s", num_cores=NC,
             num_subcores=NS)


def _fill(ref, nrows, ncols, val):
    v = jnp.full((16,), val, jnp.float32)

    @pl.loop(0, nrows)
    def _(i):
        for j in range(ncols // 16):
            ref[i, pl.ds(j * 16, 16)] = v


def _remap_all(idx_v, c, n):
    base = c * HN

    @pl.loop(0, n // 16)
    def _(v):
        iv = idx_v[pl.ds(v * 16, 16)]
        rel = iv - base
        ok = (rel >= 0) & (rel < HN)
        idx_v[pl.ds(v * 16, 16)] = jnp.where(ok, rel, HN)


CHD = 200
NCH_D = EPS // CHD


def _sc_degree(dst):
    @functools.partial(
        pl.kernel,
        out_type=jax.ShapeDtypeStruct((NPAD, 16), jnp.float32),
        mesh=plsc.VectorSubcoreMesh(**_MESH),
        scratch_types=[
            pltpu.VMEM_SHARED((HNP, 16), jnp.float32),
            pltpu.VMEM((EPS,), jnp.int32),
            pltpu.VMEM((CHD, 16), jnp.float32),
        ],
    )
    def k(dst_hbm, out_hbm, acc_sh, idx_v, ones_v):
        c = lax.axis_index("c")
        s = lax.axis_index("s")
        _fill(ones_v, CHD, 16, 0.0)
        pltpu.sync_copy(ones_v, acc_sh.at[pl.ds(s * RPT, CHD)])
        pltpu.sync_copy(ones_v.at[pl.ds(0, RPT - CHD)],
                        acc_sh.at[pl.ds(s * RPT + CHD, RPT - CHD)])
        _fill(ones_v, CHD, 16, 1.0)
        pltpu.sync_copy(dst_hbm.at[pl.ds(s * EPS, EPS)], idx_v)
        _remap_all(idx_v, c, EPS)
        plsc.subcore_barrier()

        @pl.loop(0, NCH_D)
        def _(g):
            pltpu.sync_copy(ones_v, acc_sh.at[idx_v.at[pl.ds(g * CHD, CHD)]],
                            add=True)

        plsc.subcore_barrier()
        pltpu.sync_copy(acc_sh.at[pl.ds(s * RPT, RPT)],
                        out_hbm.at[pl.ds(c * HN + s * RPT, RPT)])

    return k(dst)


@functools.cache
def _sc_conv_kernel():
    @functools.partial(
        pl.kernel,
        out_type=jax.ShapeDtypeStruct((NPAD, H), jnp.float32),
        mesh=plsc.VectorSubcoreMesh(**_MESH),
        scratch_types=[
            pltpu.VMEM_SHARED((HNP, H), jnp.float32),
            pltpu.VMEM((EPS,), jnp.int32),
            pltpu.VMEM((EPS,), jnp.int32),
            pltpu.VMEM((CHC, H), jnp.float32),
            pltpu.VMEM((CHC, H), jnp.float32),
            pltpu.SemaphoreType.DMA,
            pltpu.SemaphoreType.DMA,
        ],
    )
    def k(hs_hbm, src_hbm, dst_hbm, out_hbm, acc_sh, src_v, dst_v,
          r0, r1, s0, s1):
        c = lax.axis_index("c")
        s = lax.axis_index("s")
        rows = (r0, r1)
        sems = (s0, s1)
        _fill(r0, CHC, H, 0.0)
        for t in range(RPT // CHC):
            pltpu.sync_copy(r0, acc_sh.at[pl.ds(s * RPT + t * CHC, CHC)])
        pltpu.sync_copy(src_hbm.at[pl.ds(s * EPS, EPS)], src_v)
        pltpu.sync_copy(dst_hbm.at[pl.ds(s * EPS, EPS)], dst_v)
        _remap_all(dst_v, c, EPS)
        plsc.subcore_barrier()

        for b in range(2):
            pltpu.async_copy(hs_hbm.at[src_v.at[pl.ds(b * CHC, CHC)]],
                             rows[b], sems[b])

        @pl.loop(0, NCH_C // 2)
        def _(p):
            for b in range(2):
                cur = p * 2 + b
                pltpu.make_async_copy(
                    hs_hbm.at[src_v.at[pl.ds(cur * CHC, CHC)]],
                    rows[b], sems[b]).wait()
                pltpu.sync_copy(
                    rows[b], acc_sh.at[dst_v.at[pl.ds(cur * CHC, CHC)]],
                    add=True)
                nxt = cur + 2

                @pl.when(nxt < NCH_C)
                def _():
                    pltpu.async_copy(
                        hs_hbm.at[src_v.at[pl.ds(nxt * CHC, CHC)]],
                        rows[b], sems[b])

        plsc.subcore_barrier()
        pltpu.sync_copy(acc_sh.at[pl.ds(s * RPT, RPT)],
                        out_hbm.at[pl.ds(c * HN + s * RPT, RPT)])

    return k


def _sc_conv(hs, src, dst):
    return _sc_conv_kernel()(hs, src, dst)


NW = NC * NS
EPW = E // NW
CHE = 40
NCHE = EPW // CHE


def _sc_edge(a, b, src, dst, w2):
    @functools.partial(
        pl.kernel,
        out_type=jax.ShapeDtypeStruct((E, 16), jnp.float32),
        mesh=plsc.VectorSubcoreMesh(**_MESH),
        scratch_types=[
            pltpu.VMEM((EPW,), jnp.int32),
            pltpu.VMEM((EPW,), jnp.int32),
            pltpu.VMEM((CHE, H), jnp.float32),
            pltpu.VMEM((CHE, H), jnp.float32),
            pltpu.VMEM((CHE, H), jnp.float32),
            pltpu.VMEM((CHE, H), jnp.float32),
            pltpu.VMEM((2 * CHE, 16), jnp.float32),
            pltpu.VMEM((H,), jnp.float32),
            pltpu.SemaphoreType.DMA,
            pltpu.SemaphoreType.DMA,
        ],
    )
    def k(a_hbm, b_hbm, src_hbm, dst_hbm, w_hbm, out_hbm, src_v, dst_v,
          a0, a1, b0, b1, part_v, w_v, s0, s1):
        c = lax.axis_index("c")
        s = lax.axis_index("s")
        wid = s * NC + c
        abufs = (a0, a1)
        bbufs = (b0, b1)
        sems = (s0, s1)
        pltpu.sync_copy(w_hbm, w_v)
        pltpu.sync_copy(src_hbm.at[pl.ds(wid * EPW, EPW)], src_v)
        pltpu.sync_copy(dst_hbm.at[pl.ds(wid * EPW, EPW)], dst_v)
        wvecs = [w_v[pl.ds(j * 16, 16)] for j in range(H // 16)]

        for q in range(2):
            pltpu.async_copy(a_hbm.at[src_v.at[pl.ds(q * CHE, CHE)]],
                             abufs[q], sems[q])
            pltpu.async_copy(b_hbm.at[dst_v.at[pl.ds(q * CHE, CHE)]],
                             bbufs[q], sems[q])

        @pl.loop(0, NCHE // 2)
        def _(p):
            for q in range(2):
                cur = p * 2 + q
                pltpu.make_async_copy(
                    a_hbm.at[src_v.at[pl.ds(cur * CHE, CHE)]],
                    abufs[q], sems[q]).wait()
                pltpu.make_async_copy(
                    b_hbm.at[dst_v.at[pl.ds(cur * CHE, CHE)]],
                    bbufs[q], sems[q]).wait()

                @pl.loop(0, CHE)
                def _(e):
                    acc = jnp.zeros((16,), jnp.float32)
                    for j in range(H // 16):
                        av = abufs[q][e, pl.ds(j * 16, 16)]
                        bv = bbufs[q][e, pl.ds(j * 16, 16)]
                        acc = acc + jnp.maximum(av + bv, 0.0) * wvecs[j]
                    part_v[e + q * CHE, :] = acc

                nxt = cur + 2

                @pl.when(nxt < NCHE)
                def _():
                    pltpu.async_copy(
                        a_hbm.at[src_v.at[pl.ds(nxt * CHE, CHE)]],
                        abufs[q], sems[q])
                    pltpu.async_copy(
                        b_hbm.at[dst_v.at[pl.ds(nxt * CHE, CHE)]],
                        bbufs[q], sems[q])

            pltpu.sync_copy(part_v,
                            out_hbm.at[pl.ds(wid * EPW + p * 2 * CHE,
                                             2 * CHE)])

    return k(a, b, src, dst, w2)


_BN = 1024
_NB = NPAD // _BN


def _tc_linear(x, w, bias=None, scale=None, relu=False):
    in_specs = [
        pl.BlockSpec((_BN, x.shape[1]), lambda i: (i, 0)),
        pl.BlockSpec(w.shape, lambda i: (0, 0)),
    ]
    args = [x, w]
    if bias is not None:
        in_specs.append(pl.BlockSpec((1, H), lambda i: (0, 0)))
        args.append(bias)
    if scale is not None:
        in_specs.append(pl.BlockSpec((_BN, 1), lambda i: (i, 0)))
        args.append(scale)

    def body(*refs):
        x_ref, w_ref, rest = refs[0], refs[1], list(refs[2:-1])
        o_ref = refs[-1]
        y = jnp.dot(x_ref[...], w_ref[...], preferred_element_type=jnp.float32)
        if bias is not None:
            y = y + rest.pop(0)[...]
        if scale is not None:
            y = y * rest.pop(0)[...]
        if relu:
            y = jnp.maximum(y, 0.0)
        o_ref[...] = y

    return pl.pallas_call(
        body,
        grid=(_NB,),
        in_specs=in_specs,
        out_specs=pl.BlockSpec((_BN, H), lambda i: (i, 0)),
        out_shape=jax.ShapeDtypeStruct((NPAD, H), jnp.float32),
    )(*args)


def _tc_dinv(deg):
    def body(d_ref, o_ref):
        o_ref[...] = lax.rsqrt(1.0 + d_ref[:, :1])

    return pl.pallas_call(
        body,
        grid=(_NB,),
        in_specs=[pl.BlockSpec((_BN, 16), lambda i: (i, 0))],
        out_specs=pl.BlockSpec((_BN, 1), lambda i: (i, 0)),
        out_shape=jax.ShapeDtypeStruct((NPAD, 1), jnp.float32),
    )(deg)


def _tc_merge(acc, hs, dinv, bias):
    def body(a_ref, hs_ref, s_ref, b_ref, o_ref):
        y = s_ref[...] * (a_ref[...] + hs_ref[...]) + b_ref[...]
        o_ref[...] = jnp.maximum(y, 0.0)

    blk = lambda i: (i, 0)
    return pl.pallas_call(
        body,
        grid=(_NB,),
        in_specs=[pl.BlockSpec((_BN, H), blk), pl.BlockSpec((_BN, H), blk),
                  pl.BlockSpec((_BN, 1), blk),
                  pl.BlockSpec((1, H), lambda i: (0, 0))],
        out_specs=pl.BlockSpec((_BN, H), blk),
        out_shape=jax.ShapeDtypeStruct((NPAD, H), jnp.float32),
    )(acc, hs, dinv, bias)


_BE = 4000


def _tc_finish(part, b2):
    def body(p_ref, b_ref, o_ref):
        o_ref[...] = jnp.sum(p_ref[...], axis=-1, keepdims=True) + b_ref[...]

    return pl.pallas_call(
        body,
        grid=(E // _BE,),
        in_specs=[pl.BlockSpec((_BE, 16), lambda i: (i, 0)),
                  pl.BlockSpec((1, 1), lambda i: (0, 0))],
        out_specs=pl.BlockSpec((_BE, 1), lambda i: (i, 0)),
        out_shape=jax.ShapeDtypeStruct((E, 1), jnp.float32),
    )(part, b2)


def kernel(x, edge_index, fc_in_W, fc_in_b, conv1_W, conv1_b, conv2_W,
           conv2_b, mlp1_W, mlp1_b, mlp2_W, mlp2_b):
    src = edge_index[0]
    dst = edge_index[1]
    xp = jnp.pad(x, ((0, NPAD - N), (0, 0)))

    h0 = _tc_linear(xp, fc_in_W, bias=fc_in_b.reshape(1, H), relu=True)

    deg = _sc_degree(dst)
    dinv = _tc_dinv(deg)

    hs1 = _tc_linear(h0, conv1_W, scale=dinv)
    acc1 = _sc_conv(hs1, src, dst)
    h1 = _tc_merge(acc1, hs1, dinv, conv1_b.reshape(1, H))

    hs2 = _tc_linear(h1, conv2_W, scale=dinv)
    acc2 = _sc_conv(hs2, src, dst)
    h2 = _tc_merge(acc2, hs2, dinv, conv2_b.reshape(1, H))

    a = _tc_linear(h2, mlp1_W[:H], bias=mlp1_b.reshape(1, H))
    b = _tc_linear(h2, mlp1_W[H:])

    part = _sc_edge(a, b, src, dst, mlp2_W[:, 0])
    logits = _tc_finish(part, mlp2_b.reshape(1, 1))
    return logits[:, 0]
